# baseline (speedup 1.0000x reference)
import jax
import jax.numpy as jnp
from jax.experimental import pallas as pl
from jax.experimental.pallas import tpu as pltpu

NPOS = 8192
NPRIM = 1024
NORB = 256
BN = 512

_LOG2E = 1.4426950408889634


def _basis_block(f_ref, b_ref, oi_ref, out_ref):
    f = f_ref[...]
    g = jax.lax.dot(f, b_ref[...],
                    preferred_element_type=jnp.float32)
    px = g[:, 0 * NPRIM:1 * NPRIM]
    py = g[:, 1 * NPRIM:2 * NPRIM]
    pz = g[:, 2 * NPRIM:3 * NPRIM]
    earg = g[:, 3 * NPRIM:4 * NPRIM]

    prim = (px * py) * (pz * jnp.exp2(earg))

    col = jax.lax.broadcasted_iota(jnp.int32, (NORB, NPRIM), 0)
    s = (col == oi_ref[...]).astype(jnp.float32)
    out_ref[...] = jax.lax.dot_general(
        prim, s, (((1,), (1,)), ((), ())),
        preferred_element_type=jnp.float32)


@jax.jit
def kernel(pos, coefficients, center, alpha, norm, lmn, orbital_index):
    cn = coefficients * norm
    s0 = (lmn == 0).astype(jnp.float32)
    s1 = (lmn == 1).astype(jnp.float32)
    s2 = (lmn == 2).astype(jnp.float32)
    c = center
    b0 = s0 - s1 * c + s2 * c * c
    b1 = s1 - 2.0 * s2 * c
    b2 = s2
    b0 = b0.at[:, 0].mul(cn)
    b1 = b1.at[:, 0].mul(cn)
    b2 = b2.at[:, 0].mul(cn)
    at = alpha * _LOG2E
    e0 = -at * jnp.sum(c * c, axis=1)
    e1 = 2.0 * at[:, None] * c
    e2 = -at

    zero = jnp.zeros((NPRIM,), jnp.float32)
    def col_block(r0, rx, ry, rz, rxx, ryy, rzz):
        return jnp.stack([r0, rx, ry, rz, rxx, ryy, rzz, zero], axis=0)
    bx = col_block(b0[:, 0], b1[:, 0], zero, zero, b2[:, 0], zero, zero)
    by = col_block(b0[:, 1], zero, b1[:, 1], zero, zero, b2[:, 1], zero)
    bz = col_block(b0[:, 2], zero, zero, b1[:, 2], zero, zero, b2[:, 2])
    be = col_block(e0, e1[:, 0], e1[:, 1], e1[:, 2], e2, e2, e2)
    b = jnp.concatenate([bx, by, bz, be], axis=1)

    ones = jnp.ones((NPOS, 1), jnp.float32)
    zeros = jnp.zeros((NPOS, 1), jnp.float32)
    f = jnp.concatenate([ones, pos, pos * pos, zeros], axis=1)

    oi = orbital_index.reshape(1, NPRIM)

    grid = (NPOS // BN,)
    return pl.pallas_call(
        _basis_block,
        grid=grid,
        in_specs=[
            pl.BlockSpec((BN, 8), lambda i: (i, 0)),
            pl.BlockSpec((8, 4 * NPRIM), lambda i: (0, 0)),
            pl.BlockSpec((1, NPRIM), lambda i: (0, 0)),
        ],
        out_specs=pl.BlockSpec((BN, NORB), lambda i: (i, 0)),
        out_shape=jax.ShapeDtypeStruct((NPOS, NORB), jnp.float32),
        compiler_params=pltpu.CompilerParams(
            dimension_semantics=("parallel",)),
    )(f, b, oi)


# speedup vs baseline: 1.1667x; 1.1667x over previous
import jax
import jax.numpy as jnp
from jax.experimental import pallas as pl
from jax.experimental.pallas import tpu as pltpu

NPOS = 8192
NPRIM = 1024
NORB = 256
BN = 1024

_LOG2E = 1.4426950408889634


def _basis_block(pos_ref, cn_ref, centerT_ref, at_ref, lmnT_ref, oi_ref,
                 out_ref):
    p = pos_ref[...]
    x = p[:, 0:1]
    y = p[:, 1:2]
    z = p[:, 2:3]

    cx = centerT_ref[0:1, :]
    cy = centerT_ref[1:2, :]
    cz = centerT_ref[2:3, :]

    dx = x - cx
    dy = y - cy
    dz = z - cz
    d2x = dx * dx
    d2y = dy * dy
    d2z = dz * dz
    r2 = (d2x + d2y) + d2z

    lx = lmnT_ref[0:1, :]
    ly = lmnT_ref[1:2, :]
    lz = lmnT_ref[2:3, :]
    ax = jnp.where(lx == 0, 1.0, jnp.where(lx == 1, dx, d2x))
    ay = jnp.where(ly == 0, 1.0, jnp.where(ly == 1, dy, d2y))
    az = jnp.where(lz == 0, 1.0, jnp.where(lz == 1, dz, d2z))

    ex = jnp.exp2(at_ref[...] * r2)
    prim = (cn_ref[...] * ax) * (ay * az) * ex

    col = jax.lax.broadcasted_iota(jnp.int32, (NORB, NPRIM), 0)
    s = (col == oi_ref[...]).astype(jnp.float32)
    out_ref[...] = jax.lax.dot_general(
        prim, s, (((1,), (1,)), ((), ())),
        preferred_element_type=jnp.float32)


@jax.jit
def kernel(pos, coefficients, center, alpha, norm, lmn, orbital_index):
    cn = (coefficients * norm).reshape(1, NPRIM)
    centerT = center.T
    lmnT = lmn.T
    at = (-_LOG2E * alpha).reshape(1, NPRIM)
    oi = orbital_index.reshape(1, NPRIM)

    grid = (NPOS // BN,)
    return pl.pallas_call(
        _basis_block,
        grid=grid,
        in_specs=[
            pl.BlockSpec((BN, 3), lambda i: (i, 0)),
            pl.BlockSpec((1, NPRIM), lambda i: (0, 0)),
            pl.BlockSpec((3, NPRIM), lambda i: (0, 0)),
            pl.BlockSpec((1, NPRIM), lambda i: (0, 0)),
            pl.BlockSpec((3, NPRIM), lambda i: (0, 0)),
            pl.BlockSpec((1, NPRIM), lambda i: (0, 0)),
        ],
        out_specs=pl.BlockSpec((BN, NORB), lambda i: (i, 0)),
        out_shape=jax.ShapeDtypeStruct((NPOS, NORB), jnp.float32),
        compiler_params=pltpu.CompilerParams(
            dimension_semantics=("parallel",)),
    )(pos, cn, centerT, at, lmnT, oi)
